# baseline (speedup 1.0000x reference)
import numpy as np
import jax
import jax.numpy as jnp
from jax import lax
from jax.experimental import pallas as pl
from jax.experimental.pallas import tpu as pltpu
from jax.experimental.pallas import tpu_sc as plsc

B = 16384
F = 26
V = 100001
D = 32

_info = plsc.get_sparse_core_info()
NC, NS = _info.num_cores, _info.num_subcores
NW = NC * NS

ROWS = B * F
RPW = ROWS // NW
IDX_COLS = 128
IDX_ROWS = RPW // IDX_COLS
CHUNK = 512
SUB = CHUNK // IDX_COLS
NCHUNK = RPW // CHUNK

_OFFS = ((np.arange(RPW, dtype=np.int64) % F) * V).astype(np.int32)
_OFFS = _OFFS.reshape(IDX_ROWS, IDX_COLS)


def _body(idx_hbm, offs_hbm, tab_hbm, out_hbm, idx_v, offs_v, rows_v, gsem):
    wid = lax.axis_index("s") * NC + lax.axis_index("c")
    ibase = wid * IDX_ROWS
    obase = wid * RPW

    pltpu.sync_copy(idx_hbm.at[pl.ds(ibase, IDX_ROWS)], idx_v)
    pltpu.sync_copy(offs_hbm, offs_v)

    def add_row(r, carry):
        for k in range(IDX_COLS // 16):
            sl = (r, pl.ds(k * 16, 16))
            idx_v[sl] = idx_v[sl] + offs_v[sl]
        return carry

    lax.fori_loop(0, IDX_ROWS, add_row, 0)

    def do_chunk(c, carry):
        cps = []
        for s in range(SUB):
            cps.append(pltpu.async_copy(
                tab_hbm.at[idx_v.at[c * SUB + s]],
                rows_v.at[pl.ds(s * IDX_COLS, IDX_COLS)],
                gsem,
            ))
        for cp in cps:
            cp.wait()
        pltpu.sync_copy(rows_v, out_hbm.at[pl.ds(obase + c * CHUNK, CHUNK)])
        return carry

    lax.fori_loop(0, NCHUNK, do_chunk, 0)


def kernel(categorical_features, tables):
    idx = categorical_features.reshape(ROWS // IDX_COLS, IDX_COLS)
    tab = tables.reshape(F * V, D)
    offs = jnp.asarray(_OFFS)
    mesh = plsc.VectorSubcoreMesh(core_axis_name="c", subcore_axis_name="s")
    out = pl.kernel(
        _body,
        mesh=mesh,
        compiler_params=pltpu.CompilerParams(use_tc_tiling_on_sc=False),
        out_type=jax.ShapeDtypeStruct((ROWS, D), jnp.float32),
        scratch_types=[
            pltpu.VMEM((IDX_ROWS, IDX_COLS), jnp.int32),
            pltpu.VMEM((IDX_ROWS, IDX_COLS), jnp.int32),
            pltpu.VMEM((CHUNK, D), jnp.float32),
            pltpu.SemaphoreType.DMA,
        ],
    )(idx, offs, tab)
    return out.reshape(B, F * D)


# speedup vs baseline: 31.2540x; 31.2540x over previous
import jax
import jax.numpy as jnp
from jax import lax
from jax.experimental import pallas as pl
from jax.experimental.pallas import tpu as pltpu
from jax.experimental.pallas import tpu_sc as plsc

B = 16384
F = 26
V = 100001
D = 32

_info = plsc.get_sparse_core_info()
NC, NS = _info.num_cores, _info.num_subcores
NW = NC * NS
HALF = B // 2


def _body(cat_hbm, tab_hbm, out_hbm, tv, idx_v, out_v):
    d = lax.axis_index("s") * NC + lax.axis_index("c")

    def do_field(f, carry):
        pltpu.sync_copy(tab_hbm.at[f, d], tv)
        pltpu.sync_copy(cat_hbm.at[f], idx_v)
        c = f * D + d
        for h in range(2):
            def gath(i, carry2):
                vidx = idx_v[pl.ds(h * HALF + i * 16, 16)]
                out_v[pl.ds(i * 16, 16)] = plsc.load_gather(tv, [vidx])
                return carry2
            lax.fori_loop(0, HALF // 16, gath, 0)
            pltpu.sync_copy(out_v, out_hbm.at[c, pl.ds(h * HALF, HALF)])
        return carry

    lax.fori_loop(0, F, do_field, 0)


def kernel(categorical_features, tables):
    catT = categorical_features.T
    tabT = tables.transpose(0, 2, 1)
    mesh = plsc.VectorSubcoreMesh(core_axis_name="c", subcore_axis_name="s")
    outT = pl.kernel(
        _body,
        mesh=mesh,
        compiler_params=pltpu.CompilerParams(needs_layout_passes=False),
        out_type=jax.ShapeDtypeStruct((F * D, B), jnp.float32),
        scratch_types=[
            pltpu.VMEM((V,), jnp.float32),
            pltpu.VMEM((B,), jnp.int32),
            pltpu.VMEM((HALF,), jnp.float32),
        ],
    )(catT, tabT)
    return outT.T
